# baseline (speedup 1.0000x reference)
import functools

import jax
import jax.numpy as jnp
from jax import lax
from jax.experimental import pallas as pl
from jax.experimental.pallas import tpu as pltpu
from jax.experimental.pallas import tpu_sc as plsc

L = 16
GS = 4


def _sc_body(B, C, D, H, W, x_hbm, dt_hbm, ht_hbm, wt_hbm, out_hbm,
             htc, wtc, dtc, mbuf, xbuf, sem):
    NC = 2
    NS = 16
    NW = NC * NS
    planes = B * C * D
    per_w = planes // NW
    cc_n = per_w // D
    HW = H * W

    wid = lax.axis_index("s") * NC + lax.axis_index("c")
    p0 = wid * per_w

    def cc_loop(cc, _):
        plane0 = p0 + cc * D
        c = (plane0 // D) % C
        pltpu.sync_copy(ht_hbm.at[c], htc)
        pltpu.sync_copy(wt_hbm.at[c], wtc)
        pltpu.sync_copy(dt_hbm.at[c], dtc)

        wtv = [wtc[pl.ds(wv * L, L)] for wv in range(W // L)]
        dtv = [dtc[pl.ds(k * L, L)] for k in range(D // L)]

        for hv in range(H // L):
            hvec = htc[pl.ds(hv * L, L)]
            for li in range(L):
                h = hv * L + li
                gh = jnp.full((L,), hvec[li], jnp.float32)
                for wv in range(W // L):
                    mbuf[pl.ds(h * W + wv * L, L)] = gh + wtv[wv]

        for g in range(D // GS):
            off = (plane0 + g * GS) * HW
            pltpu.async_copy(x_hbm.at[pl.ds(off, GS * HW)], xbuf, sem).wait()
            for dl in range(GS):
                d = g * GS + dl
                sv = jnp.full((L,), dtv[d // L][d % L], jnp.float32)

                def add_loop(j, _, dl=dl, sv=sv):
                    sl = pl.ds(dl * HW + j * L, L)
                    xbuf[sl] = xbuf[sl] + (mbuf[pl.ds(j * L, L)] + sv)
                    return 0

                lax.fori_loop(0, HW // L, add_loop, 0)
            pltpu.sync_copy(xbuf, out_hbm.at[pl.ds(off, GS * HW)])
        return 0

    lax.fori_loop(0, cc_n, cc_loop, 0)


@jax.jit
def kernel(x, depth_table, height_table, width_table):
    B, C, D, H, W = x.shape
    N = B * C * D * H * W
    xf = x.reshape(N)
    dt_t = depth_table.T
    ht_t = height_table.T
    wt_t = width_table.T

    mesh = plsc.VectorSubcoreMesh(core_axis_name="c", subcore_axis_name="s")
    body = functools.partial(_sc_body, B, C, D, H, W)
    out = pl.kernel(
        body,
        out_type=jax.ShapeDtypeStruct((N,), jnp.float32),
        mesh=mesh,
        scratch_types=[
            pltpu.VMEM((H,), jnp.float32),
            pltpu.VMEM((W,), jnp.float32),
            pltpu.VMEM((D,), jnp.float32),
            pltpu.VMEM((H * W,), jnp.float32),
            pltpu.VMEM((GS * H * W,), jnp.float32),
            pltpu.SemaphoreType.DMA,
        ],
    )(xf, dt_t, ht_t, wt_t)
    return out.reshape(B, C, D, H, W)


# speedup vs baseline: 1.3724x; 1.3724x over previous
import functools

import jax
import jax.numpy as jnp
from jax import lax
from jax.experimental import pallas as pl
from jax.experimental.pallas import tpu as pltpu
from jax.experimental.pallas import tpu_sc as plsc

L = 16
GS = 4


def _sc_body(B, C, D, H, W, x_hbm, dt_hbm, ht_hbm, wt_hbm, out_hbm,
             htc, wtc, dtc, mbuf, xb0, xb1, si0, si1, so0, so1):
    NC = 2
    NS = 16
    NW = NC * NS
    planes = B * C * D
    per_w = planes // NW
    cc_n = per_w // D
    HW = H * W
    NG = D // GS

    wid = lax.axis_index("s") * NC + lax.axis_index("c")
    p0 = wid * per_w

    xb = [xb0, xb1]
    sin = [si0, si1]
    sout = [so0, so1]

    def cc_loop(cc, _):
        plane0 = p0 + cc * D
        c = (plane0 // D) % C
        pltpu.sync_copy(ht_hbm.at[c], htc)
        pltpu.sync_copy(wt_hbm.at[c], wtc)
        pltpu.sync_copy(dt_hbm.at[c], dtc)

        wtv = [wtc[pl.ds(wv * L, L)] for wv in range(W // L)]
        dtv = [dtc[pl.ds(k * L, L)] for k in range(D // L)]

        for hv in range(H // L):
            hvec = htc[pl.ds(hv * L, L)]
            for li in range(L):
                h = hv * L + li
                gh = jnp.full((L,), hvec[li], jnp.float32)
                for wv in range(W // L):
                    mbuf[pl.ds(h * W + wv * L, L)] = gh + wtv[wv]

        def start_in(g, slot):
            off = (plane0 + g * GS) * HW
            return pltpu.async_copy(x_hbm.at[pl.ds(off, GS * HW)], xb[slot], sin[slot])

        def start_out(g, slot):
            off = (plane0 + g * GS) * HW
            return pltpu.async_copy(xb[slot], out_hbm.at[pl.ds(off, GS * HW)], sout[slot])

        for g in range(NG):
            slot = g & 1
            start_in(g, slot).wait()
            buf = xb[slot]
            for dl in range(GS):
                d = g * GS + dl
                sv = jnp.full((L,), dtv[d // L][d % L], jnp.float32)

                @plsc.parallel_loop(0, HW // L, step=1, unroll=8)
                def add_loop(j, dl=dl, sv=sv, buf=buf):
                    plsc.addupdate(buf.at[pl.ds(dl * HW + j * L, L)],
                                   mbuf[pl.ds(j * L, L)] + sv)

            start_out(g, slot).wait()
        return 0

    lax.fori_loop(0, cc_n, cc_loop, 0)


@jax.jit
def kernel(x, depth_table, height_table, width_table):
    B, C, D, H, W = x.shape
    N = B * C * D * H * W
    xf = x.reshape(N)
    dt_t = depth_table.T
    ht_t = height_table.T
    wt_t = width_table.T

    mesh = plsc.VectorSubcoreMesh(core_axis_name="c", subcore_axis_name="s")
    body = functools.partial(_sc_body, B, C, D, H, W)
    out = pl.kernel(
        body,
        out_type=jax.ShapeDtypeStruct((N,), jnp.float32),
        mesh=mesh,
        scratch_types=[
            pltpu.VMEM((H,), jnp.float32),
            pltpu.VMEM((W,), jnp.float32),
            pltpu.VMEM((D,), jnp.float32),
            pltpu.VMEM((H * W,), jnp.float32),
            pltpu.VMEM((GS * H * W,), jnp.float32),
            pltpu.VMEM((GS * H * W,), jnp.float32),
            pltpu.SemaphoreType.DMA,
            pltpu.SemaphoreType.DMA,
            pltpu.SemaphoreType.DMA,
            pltpu.SemaphoreType.DMA,
        ],
    )(xf, dt_t, ht_t, wt_t)
    return out.reshape(B, C, D, H, W)


# speedup vs baseline: 1.5081x; 1.0989x over previous
import functools

import jax
import jax.numpy as jnp
from jax import lax
from jax.experimental import pallas as pl
from jax.experimental.pallas import tpu as pltpu
from jax.experimental.pallas import tpu_sc as plsc

L = 16
GS = 4


def _sc_body(B, C, D, H, W, x_hbm, dt_hbm, ht_hbm, wt_hbm, out_hbm,
             htc, wtc, dtc, mbuf, xb0, xb1, si0, si1, so0, so1):
    NC = 2
    NS = 16
    NW = NC * NS
    planes = B * C * D
    per_w = planes // NW
    cc_n = per_w // D
    HW = H * W
    NG = D // GS

    wid = lax.axis_index("s") * NC + lax.axis_index("c")
    p0 = wid * per_w

    xb = [xb0, xb1]
    sin = [si0, si1]
    sout = [so0, so1]

    def cc_loop(cc, _):
        plane0 = p0 + cc * D
        c = (plane0 // D) % C
        pltpu.sync_copy(ht_hbm.at[c], htc)
        pltpu.sync_copy(wt_hbm.at[c], wtc)
        pltpu.sync_copy(dt_hbm.at[c], dtc)

        wtv = [wtc[pl.ds(wv * L, L)] for wv in range(W // L)]
        dtv = [dtc[pl.ds(k * L, L)] for k in range(D // L)]

        for hv in range(H // L):
            hvec = htc[pl.ds(hv * L, L)]
            for li in range(L):
                h = hv * L + li
                gh = jnp.full((L,), hvec[li], jnp.float32)
                for wv in range(W // L):
                    mbuf[pl.ds(h * W + wv * L, L)] = gh + wtv[wv]

        def start_in(g, slot):
            off = (plane0 + g * GS) * HW
            return pltpu.async_copy(x_hbm.at[pl.ds(off, GS * HW)], xb[slot], sin[slot])

        def start_out(g, slot):
            off = (plane0 + g * GS) * HW
            return pltpu.async_copy(xb[slot], out_hbm.at[pl.ds(off, GS * HW)], sout[slot])

        in_desc = {0: start_in(0, 0)}
        for g in range(NG):
            slot = g & 1
            if g + 1 < NG:
                in_desc[g + 1] = start_in(g + 1, (g + 1) & 1)
            in_desc[g].wait()
            buf = xb[slot]
            for dl in range(GS):
                d = g * GS + dl
                sv = jnp.full((L,), dtv[d // L][d % L], jnp.float32)

                @plsc.parallel_loop(0, HW // L, step=1, unroll=8)
                def add_loop(j, dl=dl, sv=sv, buf=buf):
                    plsc.addupdate(buf.at[pl.ds(dl * HW + j * L, L)],
                                   mbuf[pl.ds(j * L, L)] + sv)

            start_out(g, slot).wait()
        return 0

    lax.fori_loop(0, cc_n, cc_loop, 0)


@jax.jit
def kernel(x, depth_table, height_table, width_table):
    B, C, D, H, W = x.shape
    N = B * C * D * H * W
    xf = x.reshape(N)
    dt_t = depth_table.T
    ht_t = height_table.T
    wt_t = width_table.T

    mesh = plsc.VectorSubcoreMesh(core_axis_name="c", subcore_axis_name="s")
    body = functools.partial(_sc_body, B, C, D, H, W)
    out = pl.kernel(
        body,
        out_type=jax.ShapeDtypeStruct((N,), jnp.float32),
        mesh=mesh,
        scratch_types=[
            pltpu.VMEM((H,), jnp.float32),
            pltpu.VMEM((W,), jnp.float32),
            pltpu.VMEM((D,), jnp.float32),
            pltpu.VMEM((H * W,), jnp.float32),
            pltpu.VMEM((GS * H * W,), jnp.float32),
            pltpu.VMEM((GS * H * W,), jnp.float32),
            pltpu.SemaphoreType.DMA,
            pltpu.SemaphoreType.DMA,
            pltpu.SemaphoreType.DMA,
            pltpu.SemaphoreType.DMA,
        ],
    )(xf, dt_t, ht_t, wt_t)
    return out.reshape(B, C, D, H, W)


# speedup vs baseline: 1.5607x; 1.0349x over previous
import functools

import jax
import jax.numpy as jnp
from jax import lax
from jax.experimental import pallas as pl
from jax.experimental.pallas import tpu as pltpu
from jax.experimental.pallas import tpu_sc as plsc

L = 16
GS = 4


def _sc_body(B, C, D, H, W, x_hbm, dt_hbm, ht_hbm, wt_hbm, out_hbm,
             htc, wtc, dtc, hsbuf, ib0, ib1, ob0, ob1, si0, si1, so0, so1):
    NC = 2
    NS = 16
    NW = NC * NS
    planes = B * C * D
    per_w = planes // NW
    cc_n = per_w // D
    HW = H * W
    NG = D // GS

    wid = lax.axis_index("s") * NC + lax.axis_index("c")
    p0 = wid * per_w

    ib = [ib0, ib1]
    ob = [ob0, ob1]
    sin = [si0, si1]
    sout = [so0, so1]

    def cc_loop(cc, _):
        plane0 = p0 + cc * D
        c = (plane0 // D) % C
        pltpu.sync_copy(ht_hbm.at[c], htc)
        pltpu.sync_copy(wt_hbm.at[c], wtc)
        pltpu.sync_copy(dt_hbm.at[c], dtc)

        wtv = [wtc[pl.ds(wv * L, L)] for wv in range(W // L)]
        dtv = [dtc[pl.ds(k * L, L)] for k in range(D // L)]

        for hv in range(H // L):
            hvec = htc[pl.ds(hv * L, L)]
            for li in range(L):
                h = hv * L + li
                hsbuf[pl.ds(h * L, L)] = jnp.full((L,), hvec[li], jnp.float32)

        def start_in(g, slot):
            off = (plane0 + g * GS) * HW
            return pltpu.async_copy(x_hbm.at[pl.ds(off, GS * HW)], ib[slot], sin[slot])

        def start_out(g, slot):
            off = (plane0 + g * GS) * HW
            return pltpu.async_copy(ob[slot], out_hbm.at[pl.ds(off, GS * HW)], sout[slot])

        in_desc = {0: start_in(0, 0)}
        out_desc = {}
        for g in range(NG):
            slot = g & 1
            if g + 1 < NG:
                in_desc[g + 1] = start_in(g + 1, (g + 1) & 1)
            in_desc[g].wait()
            if g >= 2:
                out_desc[g - 2].wait()
            ibuf, obuf = ib[slot], ob[slot]
            for dl in range(GS):
                d = g * GS + dl
                sv = jnp.full((L,), dtv[d // L][d % L], jnp.float32)

                @plsc.parallel_loop(0, H, step=1, unroll=4)
                def row_loop(h, dl=dl, sv=sv, ibuf=ibuf, obuf=obuf):
                    hs = hsbuf[pl.ds(h * L, L)] + sv
                    base = dl * HW + h * W
                    for wv in range(W // L):
                        sl = pl.ds(base + wv * L, L)
                        obuf[sl] = ibuf[sl] + (hs + wtv[wv])

            out_desc[g] = start_out(g, slot)
        out_desc[NG - 2].wait()
        out_desc[NG - 1].wait()
        return 0

    lax.fori_loop(0, cc_n, cc_loop, 0)


@jax.jit
def kernel(x, depth_table, height_table, width_table):
    B, C, D, H, W = x.shape
    N = B * C * D * H * W
    xf = x.reshape(N)
    dt_t = depth_table.T
    ht_t = height_table.T
    wt_t = width_table.T

    mesh = plsc.VectorSubcoreMesh(core_axis_name="c", subcore_axis_name="s")
    body = functools.partial(_sc_body, B, C, D, H, W)
    out = pl.kernel(
        body,
        out_type=jax.ShapeDtypeStruct((N,), jnp.float32),
        mesh=mesh,
        scratch_types=[
            pltpu.VMEM((H,), jnp.float32),
            pltpu.VMEM((W,), jnp.float32),
            pltpu.VMEM((D,), jnp.float32),
            pltpu.VMEM((H * L,), jnp.float32),
            pltpu.VMEM((GS * H * W,), jnp.float32),
            pltpu.VMEM((GS * H * W,), jnp.float32),
            pltpu.VMEM((GS * H * W,), jnp.float32),
            pltpu.VMEM((GS * H * W,), jnp.float32),
            pltpu.SemaphoreType.DMA,
            pltpu.SemaphoreType.DMA,
            pltpu.SemaphoreType.DMA,
            pltpu.SemaphoreType.DMA,
        ],
    )(xf, dt_t, ht_t, wt_t)
    return out.reshape(B, C, D, H, W)


# speedup vs baseline: 3.4516x; 2.2115x over previous
import jax
import jax.numpy as jnp
from jax import lax
from jax.experimental import pallas as pl
from jax.experimental.pallas import tpu as pltpu

CB = 16


def _tc_body(dt_ref, x_ref, he_ref, ho_ref, wt_ref, out_ref):
    c0 = pl.program_id(1) * CB
    D = x_ref.shape[2]
    HR, W = he_ref.shape[1], wt_ref.shape[2]
    lane = lax.broadcasted_iota(jnp.int32, (HR, 2 * W), 1)
    for cl in range(CB):
        he = he_ref[cl]
        ho = ho_ref[cl]
        hterm = jnp.where(lane < W, he, ho)
        w2 = jnp.concatenate([wt_ref[cl], wt_ref[cl]], axis=-1)
        hw = hterm + w2
        for dl in range(D):
            s = dt_ref[dl, c0 + cl]
            out_ref[0, cl, dl] = x_ref[0, cl, dl] + (hw + s)


@jax.jit
def kernel(x, depth_table, height_table, width_table):
    B, C, D, H, W = x.shape
    xp = x.reshape(B, C, D, H // 2, 2 * W)
    ht_t = height_table.T
    he = ht_t[:, 0::2].reshape(C, H // 2, 1)
    ho = ht_t[:, 1::2].reshape(C, H // 2, 1)
    wt_t = width_table.T.reshape(C, 1, W)

    grid = (B, C // CB)
    out = pl.pallas_call(
        _tc_body,
        grid=grid,
        in_specs=[
            pl.BlockSpec(memory_space=pltpu.SMEM),
            pl.BlockSpec((1, CB, D, H // 2, 2 * W), lambda b, c: (b, c, 0, 0, 0)),
            pl.BlockSpec((CB, H // 2, 1), lambda b, c: (c, 0, 0)),
            pl.BlockSpec((CB, H // 2, 1), lambda b, c: (c, 0, 0)),
            pl.BlockSpec((CB, 1, W), lambda b, c: (c, 0, 0)),
        ],
        out_specs=pl.BlockSpec((1, CB, D, H // 2, 2 * W), lambda b, c: (b, c, 0, 0, 0)),
        out_shape=jax.ShapeDtypeStruct(xp.shape, x.dtype),
    )(depth_table, xp, he, ho, wt_t)
    return out.reshape(B, C, D, H, W)
